# baseline (speedup 1.0000x reference)
import functools

import jax
import jax.numpy as jnp
from jax import lax
from jax.experimental import pallas as pl
from jax.experimental.pallas import tpu as pltpu
from jax.experimental.pallas import tpu_sc as plsc

NEG = 0.2
EPS = 1e-8
B, T, N, C, E = 2, 2, 10000, 64, 160000
BT = B * T
NP = 10240
NS = 16
NC = 2
CH = 128
NCH = 80
EPAD = NS * NCH * CH
ROWS_PT = NP // NS
WCH = 80
FBLK = 512


def _feat_body(x_ref, a_ref, f1_ref, f2_ref):
    f = jnp.dot(x_ref[...], a_ref[...], preferred_element_type=jnp.float32)
    f1_ref[...] = f[:, 0]
    f2_ref[...] = f[:, 1]


def _feats(x2d, a12):
    return pl.pallas_call(
        _feat_body,
        grid=(x2d.shape[0] // FBLK,),
        in_specs=[
            pl.BlockSpec((FBLK, C), lambda i: (i, 0)),
            pl.BlockSpec((C, 2), lambda i: (0, 0)),
        ],
        out_specs=[
            pl.BlockSpec((FBLK,), lambda i: (i,)),
            pl.BlockSpec((FBLK,), lambda i: (i,)),
        ],
        out_shape=[
            jax.ShapeDtypeStruct((x2d.shape[0],), jnp.float32),
            jax.ShapeDtypeStruct((x2d.shape[0],), jnp.float32),
        ],
    )(x2d, a12)


def _sc_body(f1_hbm, f2_hbm, dst_hbm, src_hbm, x_hbm, out_hbm,
             f1_loc, f2_loc, rden_loc, dstv, srcv, expev, gbuf, sbuf, rvbuf,
             semg0, semg1, sems0, sems1, semd,
             den_sp, out_sp):
    c = lax.axis_index("c")
    s = lax.axis_index("s")
    zeros16 = jnp.zeros((16,), jnp.float32)
    iota16 = lax.iota(jnp.int32, 16)
    semg = (semg0, semg1)
    sems = (sems0, sems1)

    pltpu.sync_copy(dst_hbm.at[s], dstv)
    pltpu.sync_copy(src_hbm.at[s], srcv)

    for bti in range(2):
        bt = c * 2 + bti

        @pl.loop(0, CH)
        def _zero_rows(e):
            for v in range(4):
                sbuf[0, e, pl.ds(v * 16, 16)] = zeros16

        @pl.loop(0, ROWS_PT // 16)
        def _zero_den(g):
            rden_loc[pl.ds(g * 16, 16)] = zeros16

        for q in range(ROWS_PT // CH):
            pltpu.sync_copy(sbuf.at[0],
                            out_sp.at[pl.ds(s * ROWS_PT + q * CH, CH), :])
        pltpu.sync_copy(rden_loc.at[pl.ds(0, ROWS_PT)],
                        den_sp.at[pl.ds(s * ROWS_PT, ROWS_PT)])

        pltpu.sync_copy(f1_hbm.at[bt], f1_loc)
        pltpu.sync_copy(f2_hbm.at[bt], f2_loc)

        plsc.subcore_barrier()

        def _slot(j, b, guard_next):
            ob = 1 - b
            jn = j + 1

            def _issue_next():
                pltpu.async_copy(x_hbm.at[bt].at[srcv.at[jn]],
                                 gbuf.at[ob], semg[ob])
            if guard_next:
                pl.when(jn < NCH)(_issue_next)
            else:
                _issue_next()

            for k in range(CH // 16):
                d16 = dstv[j, pl.ds(k * 16, 16)]
                s16 = srcv[j, pl.ds(k * 16, 16)]
                e = (plsc.load_gather(f1_loc, [d16])
                     + plsc.load_gather(f2_loc, [s16]))
                e = jnp.where(e >= 0.0, e, NEG * e)
                expev[j, pl.ds(k * 16, 16)] = jnp.exp(e)
            pltpu.async_copy(expev.at[j], den_sp.at[dstv.at[j]], semd,
                             add=True)

            pltpu.make_async_copy(x_hbm.at[bt].at[srcv.at[j]],
                                  gbuf.at[b], semg[b]).wait()

            @pl.when(j >= 2)
            def _drain_prev():
                pltpu.make_async_copy(sbuf.at[b],
                                      out_sp.at[dstv.at[j - 2]],
                                      sems[b]).wait()

            @pl.loop(0, CH // 16)
            def _scale(g2):
                for r in range(16):
                    e2 = g2 * 16 + r
                    spl = plsc.load_gather(
                        expev, [jnp.full((16,), j, jnp.int32),
                                jnp.full((16,), e2, jnp.int32)])
                    for v in range(4):
                        sbuf[b, e2, pl.ds(v * 16, 16)] = (
                            gbuf[b, e2, pl.ds(v * 16, 16)] * spl)

            pltpu.async_copy(sbuf.at[b], out_sp.at[dstv.at[j]], sems[b],
                             add=True)

        pltpu.async_copy(x_hbm.at[bt].at[srcv.at[0]], gbuf.at[0], semg[0])

        @pl.loop(0, NCH // 2)
        def _edge_pair(g):
            _slot(2 * g, 0, guard_next=False)
            _slot(2 * g + 1, 1, guard_next=True)

        pltpu.make_async_copy(sbuf.at[0], out_sp.at[dstv.at[NCH - 2]],
                              sems[0]).wait()
        pltpu.make_async_copy(sbuf.at[1], out_sp.at[dstv.at[NCH - 1]],
                              sems[1]).wait()

        @pl.loop(0, NCH)
        def _drain_den(jj):
            pltpu.make_async_copy(expev.at[jj], den_sp.at[dstv.at[jj]],
                                  semd).wait()

        plsc.subcore_barrier()

        pltpu.sync_copy(den_sp.at[pl.ds(s * ROWS_PT, ROWS_PT)], rden_loc)
        nch_out = jnp.where(s == NS - 1, (N - (NS - 1) * ROWS_PT) // WCH,
                            ROWS_PT // WCH)

        @pl.loop(0, nch_out)
        def _writeback(cc):
            r0 = s * ROWS_PT + cc * WCH
            pltpu.sync_copy(out_sp.at[pl.ds(r0, WCH), :],
                            sbuf.at[0, pl.ds(0, WCH), :])
            for rg in range(WCH // 16):
                dv = plsc.load_gather(rden_loc, [iota16 + (cc * WCH + rg * 16)])
                rvbuf[pl.ds(rg * 16, 16)] = 1.0 / jnp.maximum(dv, EPS)

            @pl.loop(0, WCH, unroll=4)
            def _scale_out(rr):
                spl = plsc.load_gather(rvbuf, [jnp.full((16,), rr, jnp.int32)])
                for v in range(4):
                    sbuf[0, rr, pl.ds(v * 16, 16)] = (
                        sbuf[0, rr, pl.ds(v * 16, 16)] * spl)

            pltpu.sync_copy(sbuf.at[0, pl.ds(0, WCH), :],
                            out_hbm.at[bt].at[pl.ds(r0, WCH), :])

        plsc.subcore_barrier()


_sc_kernel = functools.partial(
    pl.kernel,
    out_type=jax.ShapeDtypeStruct((BT, N, C), jnp.float32),
    mesh=plsc.VectorSubcoreMesh(core_axis_name="c", subcore_axis_name="s"),
    compiler_params=pltpu.CompilerParams(needs_layout_passes=False,
                                         use_tc_tiling_on_sc=False),
    scratch_types=[
        pltpu.VMEM((NP,), jnp.float32),
        pltpu.VMEM((NP,), jnp.float32),
        pltpu.VMEM((ROWS_PT,), jnp.float32),
        pltpu.VMEM((NCH, CH), jnp.int32),
        pltpu.VMEM((NCH, CH), jnp.int32),
        pltpu.VMEM((NCH, CH), jnp.float32),
        pltpu.VMEM((2, CH, C), jnp.float32),
        pltpu.VMEM((2, CH, C), jnp.float32),
        pltpu.VMEM((WCH,), jnp.float32),
        pltpu.SemaphoreType.DMA,
        pltpu.SemaphoreType.DMA,
        pltpu.SemaphoreType.DMA,
        pltpu.SemaphoreType.DMA,
        pltpu.SemaphoreType.DMA,
        pltpu.VMEM_SHARED((NP,), jnp.float32),
        pltpu.VMEM_SHARED((NP, C), jnp.float32),
    ],
)(_sc_body)


def kernel(x_BTNC, edge_index, a):
    x = x_BTNC.reshape(BT, N, C)
    x_pad = jnp.pad(x, ((0, 0), (0, NP - N), (0, 0)))
    a12 = jnp.concatenate([a[:C], a[C:]], axis=1)
    feat1, feat2 = _feats(x_pad.reshape(BT * NP, C), a12)

    dst = edge_index[0].astype(jnp.int32)
    src = edge_index[1].astype(jnp.int32)
    npad = EPAD - E
    pad_dst = N + (jnp.arange(npad, dtype=jnp.int32) % (NP - N))
    pad_src = jnp.arange(npad, dtype=jnp.int32) % N
    dst3 = jnp.concatenate([dst, pad_dst]).reshape(NS, NCH, CH)
    src3 = jnp.concatenate([src, pad_src]).reshape(NS, NCH, CH)

    out = _sc_kernel(feat1.reshape(BT, NP), feat2.reshape(BT, NP),
                     dst3, src3, x_pad)
    return out.reshape(B, T, N, C)


# speedup vs baseline: 2.0259x; 2.0259x over previous
import functools

import jax
import jax.numpy as jnp
from jax import lax
from jax.experimental import pallas as pl
from jax.experimental.pallas import tpu as pltpu
from jax.experimental.pallas import tpu_sc as plsc

NEG = 0.2
EPS = 1e-8
B, T, N, C, E = 2, 2, 10000, 64, 160000
BT = B * T
NP = 10240
NS = 16
NC = 2
CH = 128
NCH = 80
EPAD = NS * NCH * CH
ROWS_PT = NP // NS
WCH = 80
FBLK = 512


def _feat_body(x_ref, a_ref, f1_ref, f2_ref):
    f = jnp.dot(x_ref[...], a_ref[...], preferred_element_type=jnp.float32)
    f1_ref[...] = f[:, 0]
    f2_ref[...] = f[:, 1]


def _feats(x2d, a12):
    return pl.pallas_call(
        _feat_body,
        grid=(x2d.shape[0] // FBLK,),
        in_specs=[
            pl.BlockSpec((FBLK, C), lambda i: (i, 0)),
            pl.BlockSpec((C, 2), lambda i: (0, 0)),
        ],
        out_specs=[
            pl.BlockSpec((FBLK,), lambda i: (i,)),
            pl.BlockSpec((FBLK,), lambda i: (i,)),
        ],
        out_shape=[
            jax.ShapeDtypeStruct((x2d.shape[0],), jnp.float32),
            jax.ShapeDtypeStruct((x2d.shape[0],), jnp.float32),
        ],
    )(x2d, a12)


def _sc_body(f1_hbm, f2_hbm, dst_hbm, src_hbm, x_hbm, out_hbm,
             f1_loc, f2_loc, rden_loc, dstv, srcv, expev, gbuf, sbuf, rvbuf,
             semg0, semg1, sems0, sems1, semd,
             den_sp, out_sp):
    c = lax.axis_index("c")
    s = lax.axis_index("s")
    zeros16 = jnp.zeros((16,), jnp.float32)
    iota16 = lax.iota(jnp.int32, 16)
    semg = (semg0, semg1)
    sems = (sems0, sems1)

    pltpu.sync_copy(dst_hbm.at[s], dstv)
    pltpu.sync_copy(src_hbm.at[s], srcv)

    for bti in range(2):
        bt = c * 2 + bti

        @pl.loop(0, CH)
        def _zero_rows(e):
            for v in range(4):
                sbuf[0, e, pl.ds(v * 16, 16)] = zeros16

        @pl.loop(0, ROWS_PT // 16)
        def _zero_den(g):
            rden_loc[pl.ds(g * 16, 16)] = zeros16

        for q in range(ROWS_PT // CH):
            pltpu.sync_copy(sbuf.at[0],
                            out_sp.at[pl.ds(s * ROWS_PT + q * CH, CH), :])
        pltpu.sync_copy(rden_loc.at[pl.ds(0, ROWS_PT)],
                        den_sp.at[pl.ds(s * ROWS_PT, ROWS_PT)])

        pltpu.sync_copy(f1_hbm.at[bt], f1_loc)
        pltpu.sync_copy(f2_hbm.at[bt], f2_loc)

        plsc.subcore_barrier()

        def _slot(j, b, guard_next):
            ob = 1 - b
            jn = j + 1

            def _issue_next():
                pltpu.async_copy(x_hbm.at[bt].at[srcv.at[jn]],
                                 gbuf.at[ob], semg[ob])
            if guard_next:
                pl.when(jn < NCH)(_issue_next)
            else:
                _issue_next()

            for k in range(CH // 16):
                d16 = dstv[j, pl.ds(k * 16, 16)]
                s16 = srcv[j, pl.ds(k * 16, 16)]
                e = (plsc.load_gather(f1_loc, [d16])
                     + plsc.load_gather(f2_loc, [s16]))
                e = jnp.where(e >= 0.0, e, NEG * e)
                expev[j, pl.ds(k * 16, 16)] = jnp.exp(e)
            pltpu.async_copy(expev.at[j], den_sp.at[dstv.at[j]], semd,
                             add=True)

            pltpu.make_async_copy(x_hbm.at[bt].at[srcv.at[j]],
                                  gbuf.at[b], semg[b]).wait()

            @pl.when(j >= 2)
            def _drain_prev():
                pltpu.make_async_copy(sbuf.at[b],
                                      out_sp.at[dstv.at[j - 2]],
                                      sems[b]).wait()

            @pl.loop(0, CH // 16)
            def _scale(g2):
                ev = expev[j, pl.ds(g2 * 16, 16)]
                for r in range(16):
                    e2 = g2 * 16 + r
                    spl = ev[r]
                    for v in range(4):
                        sbuf[b, e2, pl.ds(v * 16, 16)] = (
                            gbuf[b, e2, pl.ds(v * 16, 16)] * spl)

            pltpu.async_copy(sbuf.at[b], out_sp.at[dstv.at[j]], sems[b],
                             add=True)

        pltpu.async_copy(x_hbm.at[bt].at[srcv.at[0]], gbuf.at[0], semg[0])

        @pl.loop(0, NCH // 2)
        def _edge_pair(g):
            _slot(2 * g, 0, guard_next=False)
            _slot(2 * g + 1, 1, guard_next=True)

        pltpu.make_async_copy(sbuf.at[0], out_sp.at[dstv.at[NCH - 2]],
                              sems[0]).wait()
        pltpu.make_async_copy(sbuf.at[1], out_sp.at[dstv.at[NCH - 1]],
                              sems[1]).wait()

        @pl.loop(0, NCH)
        def _drain_den(jj):
            pltpu.make_async_copy(expev.at[jj], den_sp.at[dstv.at[jj]],
                                  semd).wait()

        plsc.subcore_barrier()

        pltpu.sync_copy(den_sp.at[pl.ds(s * ROWS_PT, ROWS_PT)], rden_loc)
        nch_out = jnp.where(s == NS - 1, (N - (NS - 1) * ROWS_PT) // WCH,
                            ROWS_PT // WCH)

        @pl.loop(0, nch_out)
        def _writeback(cc):
            r0 = s * ROWS_PT + cc * WCH
            pltpu.sync_copy(out_sp.at[pl.ds(r0, WCH), :],
                            sbuf.at[0, pl.ds(0, WCH), :])
            for rg in range(WCH // 16):
                dv = plsc.load_gather(rden_loc, [iota16 + (cc * WCH + rg * 16)])
                rv = 1.0 / jnp.maximum(dv, EPS)
                for r in range(16):
                    rr = rg * 16 + r
                    spl = rv[r]
                    for v in range(4):
                        sbuf[0, rr, pl.ds(v * 16, 16)] = (
                            sbuf[0, rr, pl.ds(v * 16, 16)] * spl)

            pltpu.sync_copy(sbuf.at[0, pl.ds(0, WCH), :],
                            out_hbm.at[bt].at[pl.ds(r0, WCH), :])

        plsc.subcore_barrier()


_sc_kernel = functools.partial(
    pl.kernel,
    out_type=jax.ShapeDtypeStruct((BT, N, C), jnp.float32),
    mesh=plsc.VectorSubcoreMesh(core_axis_name="c", subcore_axis_name="s"),
    compiler_params=pltpu.CompilerParams(needs_layout_passes=False,
                                         use_tc_tiling_on_sc=False),
    scratch_types=[
        pltpu.VMEM((NP,), jnp.float32),
        pltpu.VMEM((NP,), jnp.float32),
        pltpu.VMEM((ROWS_PT,), jnp.float32),
        pltpu.VMEM((NCH, CH), jnp.int32),
        pltpu.VMEM((NCH, CH), jnp.int32),
        pltpu.VMEM((NCH, CH), jnp.float32),
        pltpu.VMEM((2, CH, C), jnp.float32),
        pltpu.VMEM((2, CH, C), jnp.float32),
        pltpu.VMEM((WCH,), jnp.float32),
        pltpu.SemaphoreType.DMA,
        pltpu.SemaphoreType.DMA,
        pltpu.SemaphoreType.DMA,
        pltpu.SemaphoreType.DMA,
        pltpu.SemaphoreType.DMA,
        pltpu.VMEM_SHARED((NP,), jnp.float32),
        pltpu.VMEM_SHARED((NP, C), jnp.float32),
    ],
)(_sc_body)


def kernel(x_BTNC, edge_index, a):
    x = x_BTNC.reshape(BT, N, C)
    x_pad = jnp.pad(x, ((0, 0), (0, NP - N), (0, 0)))
    a12 = jnp.concatenate([a[:C], a[C:]], axis=1)
    feat1, feat2 = _feats(x_pad.reshape(BT * NP, C), a12)

    dst = edge_index[0].astype(jnp.int32)
    src = edge_index[1].astype(jnp.int32)
    npad = EPAD - E
    pad_dst = N + (jnp.arange(npad, dtype=jnp.int32) % (NP - N))
    pad_src = jnp.arange(npad, dtype=jnp.int32) % N
    dst3 = jnp.concatenate([dst, pad_dst]).reshape(NS, NCH, CH)
    src3 = jnp.concatenate([src, pad_src]).reshape(NS, NCH, CH)

    out = _sc_kernel(feat1.reshape(BT, NP), feat2.reshape(BT, NP),
                     dst3, src3, x_pad)
    return out.reshape(B, T, N, C)


# speedup vs baseline: 2.3713x; 1.1705x over previous
import functools

import jax
import jax.numpy as jnp
from jax import lax
from jax.experimental import pallas as pl
from jax.experimental.pallas import tpu as pltpu
from jax.experimental.pallas import tpu_sc as plsc

NEG = 0.2
EPS = 1e-8
B, T, N, C, E = 2, 2, 10000, 64, 160000
BT = B * T
NP = 10240
NS = 16
NC = 2
CH = 128
NCH = 80
EPAD = NS * NCH * CH
ROWS_PT = NP // NS
WCH = 80
FBLK = 512


def _feat_body(x_ref, a_ref, f1_ref, f2_ref):
    for i in range(BT):
        f = jnp.dot(x_ref[i], a_ref[...], preferred_element_type=jnp.float32)
        f1_ref[i, :] = f[:, 0]
        f2_ref[i, :] = f[:, 1]


def _feats(x3, a12):
    return pl.pallas_call(
        _feat_body,
        out_shape=[
            jax.ShapeDtypeStruct((BT, N), jnp.float32),
            jax.ShapeDtypeStruct((BT, N), jnp.float32),
        ],
    )(x3, a12)


def _sc_body(f1_hbm, f2_hbm, dst_hbm, src_hbm, x_hbm, out_hbm,
             f1_loc, f2_loc, rden_loc, dstv, srcv, expev, gbuf, sbuf, rvbuf,
             semg0, semg1, sems0, sems1, semd,
             den_sp, out_sp):
    c = lax.axis_index("c")
    s = lax.axis_index("s")
    zeros16 = jnp.zeros((16,), jnp.float32)
    iota16 = lax.iota(jnp.int32, 16)
    semg = (semg0, semg1)
    sems = (sems0, sems1)
    del rvbuf

    pltpu.sync_copy(dst_hbm.at[s], dstv)
    pltpu.sync_copy(src_hbm.at[s], srcv)

    for bti in range(2):
        bt = c * 2 + bti

        @pl.loop(0, CH)
        def _zero_rows(e):
            for v in range(4):
                sbuf[0, e, pl.ds(v * 16, 16)] = zeros16

        @pl.loop(0, ROWS_PT // 16)
        def _zero_den(g):
            rden_loc[pl.ds(g * 16, 16)] = zeros16

        for q in range(ROWS_PT // CH):
            pltpu.sync_copy(sbuf.at[0],
                            out_sp.at[pl.ds(s * ROWS_PT + q * CH, CH), :])
        pltpu.sync_copy(rden_loc.at[pl.ds(0, ROWS_PT)],
                        den_sp.at[pl.ds(s * ROWS_PT, ROWS_PT)])

        for tg in range((NP - N) // 16):
            f1_loc[pl.ds(N + tg * 16, 16)] = zeros16
            f2_loc[pl.ds(N + tg * 16, 16)] = zeros16
        pltpu.sync_copy(f1_hbm.at[bt], f1_loc.at[pl.ds(0, N)])
        pltpu.sync_copy(f2_hbm.at[bt], f2_loc.at[pl.ds(0, N)])

        plsc.subcore_barrier()

        def _slot(j, b, guard_next):
            ob = 1 - b
            jn = j + 1

            def _issue_next():
                pltpu.async_copy(x_hbm.at[bt].at[srcv.at[jn]],
                                 gbuf.at[ob], semg[ob])
            if guard_next:
                pl.when(jn < NCH)(_issue_next)
            else:
                _issue_next()

            for k in range(CH // 16):
                d16 = dstv[j, pl.ds(k * 16, 16)]
                s16 = srcv[j, pl.ds(k * 16, 16)]
                e = (plsc.load_gather(f1_loc, [d16])
                     + plsc.load_gather(f2_loc, [s16]))
                e = jnp.where(e >= 0.0, e, NEG * e)
                expev[j, pl.ds(k * 16, 16)] = jnp.exp(e)
            pltpu.async_copy(expev.at[j], den_sp.at[dstv.at[j]], semd,
                             add=True)

            pltpu.make_async_copy(x_hbm.at[bt].at[srcv.at[j]],
                                  gbuf.at[b], semg[b]).wait()

            @pl.when(j >= 2)
            def _drain_prev():
                pltpu.make_async_copy(sbuf.at[b],
                                      out_sp.at[dstv.at[j - 2]],
                                      sems[b]).wait()

            @pl.loop(0, CH // 16)
            def _scale(g2):
                ev = expev[j, pl.ds(g2 * 16, 16)]
                for r in range(16):
                    e2 = g2 * 16 + r
                    spl = ev[r]
                    for v in range(4):
                        sbuf[b, e2, pl.ds(v * 16, 16)] = (
                            gbuf[b, e2, pl.ds(v * 16, 16)] * spl)

            pltpu.async_copy(sbuf.at[b], out_sp.at[dstv.at[j]], sems[b],
                             add=True)

        pltpu.async_copy(x_hbm.at[bt].at[srcv.at[0]], gbuf.at[0], semg[0])

        @pl.loop(0, NCH // 2)
        def _edge_pair(g):
            _slot(2 * g, 0, guard_next=False)
            _slot(2 * g + 1, 1, guard_next=True)

        pltpu.make_async_copy(sbuf.at[0], out_sp.at[dstv.at[NCH - 2]],
                              sems[0]).wait()
        pltpu.make_async_copy(sbuf.at[1], out_sp.at[dstv.at[NCH - 1]],
                              sems[1]).wait()

        @pl.loop(0, NCH)
        def _drain_den(jj):
            pltpu.make_async_copy(expev.at[jj], den_sp.at[dstv.at[jj]],
                                  semd).wait()

        plsc.subcore_barrier()

        pltpu.sync_copy(den_sp.at[pl.ds(s * ROWS_PT, ROWS_PT)], rden_loc)
        nch_out = jnp.where(s == NS - 1, (N - (NS - 1) * ROWS_PT) // WCH,
                            ROWS_PT // WCH)

        @pl.loop(0, nch_out)
        def _writeback(cc):
            r0 = s * ROWS_PT + cc * WCH
            pltpu.sync_copy(out_sp.at[pl.ds(r0, WCH), :],
                            sbuf.at[0, pl.ds(0, WCH), :])
            for rg in range(WCH // 16):
                dv = plsc.load_gather(rden_loc, [iota16 + (cc * WCH + rg * 16)])
                rv = 1.0 / jnp.maximum(dv, EPS)
                for r in range(16):
                    rr = rg * 16 + r
                    spl = rv[r]
                    for v in range(4):
                        sbuf[0, rr, pl.ds(v * 16, 16)] = (
                            sbuf[0, rr, pl.ds(v * 16, 16)] * spl)

            pltpu.sync_copy(sbuf.at[0, pl.ds(0, WCH), :],
                            out_hbm.at[c, bti].at[pl.ds(r0, WCH), :])

        plsc.subcore_barrier()


_sc_kernel = functools.partial(
    pl.kernel,
    out_type=jax.ShapeDtypeStruct((B, T, N, C), jnp.float32),
    mesh=plsc.VectorSubcoreMesh(core_axis_name="c", subcore_axis_name="s"),
    compiler_params=pltpu.CompilerParams(needs_layout_passes=False,
                                         use_tc_tiling_on_sc=False),
    scratch_types=[
        pltpu.VMEM((NP,), jnp.float32),
        pltpu.VMEM((NP,), jnp.float32),
        pltpu.VMEM((ROWS_PT,), jnp.float32),
        pltpu.VMEM((NCH, CH), jnp.int32),
        pltpu.VMEM((NCH, CH), jnp.int32),
        pltpu.VMEM((NCH, CH), jnp.float32),
        pltpu.VMEM((2, CH, C), jnp.float32),
        pltpu.VMEM((2, CH, C), jnp.float32),
        pltpu.VMEM((WCH,), jnp.float32),
        pltpu.SemaphoreType.DMA,
        pltpu.SemaphoreType.DMA,
        pltpu.SemaphoreType.DMA,
        pltpu.SemaphoreType.DMA,
        pltpu.SemaphoreType.DMA,
        pltpu.VMEM_SHARED((NP,), jnp.float32),
        pltpu.VMEM_SHARED((NP, C), jnp.float32),
    ],
)(_sc_body)


def kernel(x_BTNC, edge_index, a):
    x3 = x_BTNC.reshape(BT, N, C)
    a12 = jnp.concatenate([a[:C], a[C:]], axis=1)
    feat1, feat2 = _feats(x3, a12)

    dst = edge_index[0].astype(jnp.int32)
    src = edge_index[1].astype(jnp.int32)
    npad = EPAD - E
    pad_dst = N + (jnp.arange(npad, dtype=jnp.int32) % (NP - N))
    pad_src = jnp.arange(npad, dtype=jnp.int32) % N
    dst3 = jnp.concatenate([dst, pad_dst]).reshape(NS, NCH, CH)
    src3 = jnp.concatenate([src, pad_src]).reshape(NS, NCH, CH)

    return _sc_kernel(feat1, feat2, dst3, src3, x3)
